# baseline (speedup 1.0000x reference)
import functools

import jax
import jax.numpy as jnp
from jax import lax
from jax.experimental import pallas as pl
from jax.experimental.pallas import tpu as pltpu
from jax.experimental.pallas import tpu_sc as plsc

_N_CAT = 26
_N_NUM = 13
_VOCAB = 100001
_EMBED = 32
_HIDDEN = 64
_BATCH = 16384

_GROUPS = (2, 6, 6, 6, 6)

_NLT = 784
_LCH = 12544
_NLCH = 8

_NC = 2
_NS = 16
_NW = _NC * _NS

_IDX_CH = 8192
_NCH = _BATCH // _IDX_CH
_LANES = 16


def _make_repack_body(rpb):
    def _repack_body(in_ref, out_ref):
        for r in range(rpb):
            for k in range(_LCH // 128):
                out_ref[r, k] = in_ref[8 * r:8 * (r + 1), 128 * k:128 * (k + 1)]
    return _repack_body


def _tc_repack(tab2d, cbase, cpg):
    grg = cpg * _EMBED // 8
    rpb = 8
    nblk = grg // rpb
    base = cbase * _EMBED // (8 * rpb)
    return pl.pallas_call(
        _make_repack_body(rpb),
        grid=(nblk, _NLCH),
        in_specs=[pl.BlockSpec((8 * rpb, _LCH), lambda i, j: (i + base, j))],
        out_specs=pl.BlockSpec(
            (rpb, _LCH // 128, 8, 128), lambda i, j: (i, j, 0, 0)
        ),
        out_shape=jax.ShapeDtypeStruct((grg, _NLT, 8, 128), jnp.float32),
    )(tab2d)


def _sc_pooled_embedding_t(tab_packed, idx_t, cbase, cpg):
    mesh = plsc.VectorSubcoreMesh(core_axis_name="c", subcore_axis_name="s")

    @functools.partial(
        pl.kernel,
        mesh=mesh,
        compiler_params=pltpu.CompilerParams(
            use_tc_tiling_on_sc=False, needs_layout_passes=False
        ),
        out_type=jax.ShapeDtypeStruct((_EMBED, _BATCH), jnp.float32),
        scratch_types=[
            pltpu.VMEM((_NLT, 1, 128), jnp.float32),
            pltpu.VMEM((_IDX_CH,), jnp.int32),
            pltpu.VMEM((_BATCH,), jnp.float32),
        ],
    )
    def sc_kernel(tab_hbm, idx_hbm, out_hbm, vocab_v, idx_v, acc_v):
        wid = lax.axis_index("s") * _NC + lax.axis_index("c")
        zeros = jnp.zeros((_LANES,), jnp.float32)
        izeros = jnp.zeros((_LANES,), jnp.int32)

        @pl.loop(0, _BATCH // _LANES, unroll=8)
        def _zero(j):
            acc_v[pl.ds(j * _LANES, _LANES)] = zeros

        @pl.loop(0, cpg)
        def _feature(c):
            row = c * _EMBED + wid
            rg = row // 8
            s = row % 8
            pltpu.sync_copy(tab_hbm.at[rg, :, pl.ds(s, 1), :], vocab_v)
            for ch in range(_NCH):
                pltpu.sync_copy(
                    idx_hbm.at[cbase + c, pl.ds(ch * _IDX_CH, _IDX_CH)], idx_v
                )

                @plsc.parallel_loop(0, _IDX_CH // _LANES, unroll=16)
                def _gather(j):
                    ids = idx_v[pl.ds(j * _LANES, _LANES)]
                    lb = lax.shift_right_logical(ids, 7)
                    ln = lax.bitwise_and(ids, 127)
                    vals = plsc.load_gather(vocab_v, [lb, izeros, ln])
                    off = ch * _IDX_CH + j * _LANES
                    plsc.addupdate(acc_v.at[pl.ds(off, _LANES)], vals)

        pltpu.sync_copy(acc_v, out_hbm.at[wid])

    return sc_kernel(tab_packed, idx_t)


def _dense_body(*refs):
    np_ = len(_GROUPS)
    p_refs = refs[:np_]
    num_ref, w1t_ref, w2t_ref, out_ref = refs[np_:]
    w12 = jnp.dot(w1t_ref[...], w2t_ref[...], preferred_element_type=jnp.float32)
    pt = p_refs[0][...]
    for p in p_refs[1:]:
        pt = pt + p[...]
    obj = lax.dot_general(
        pt, w2t_ref[...],
        dimension_numbers=(((0,), (0,)), ((), ())),
        preferred_element_type=jnp.float32,
    )
    out_ref[...] = obj + jnp.dot(num_ref[...], w12, preferred_element_type=jnp.float32)


def _tc_dense(pooled, num, w1t, w2t):
    blk = 2048
    grid = _BATCH // blk
    return pl.pallas_call(
        _dense_body,
        grid=(grid,),
        in_specs=[pl.BlockSpec((_EMBED, blk), lambda i: (0, i))
                  for _ in pooled]
        + [
            pl.BlockSpec((blk, _N_NUM), lambda i: (i, 0)),
            pl.BlockSpec((_N_NUM, _EMBED), lambda i: (0, 0)),
            pl.BlockSpec((_EMBED, _HIDDEN), lambda i: (0, 0)),
        ],
        out_specs=pl.BlockSpec((blk, _HIDDEN), lambda i: (i, 0)),
        out_shape=jax.ShapeDtypeStruct((_BATCH, _HIDDEN), jnp.float32),
    )(*pooled, num, w1t, w2t)


@jax.jit
def kernel(edge_feats, tables, W1, W2):
    tab2d = jnp.transpose(tables, (0, 2, 1)).reshape(_N_CAT * _EMBED, _VOCAB)
    idx_t = jnp.transpose(edge_feats[:, :_N_CAT].astype(jnp.int32), (1, 0))
    pooled = []
    cbase = 0
    for cpg in _GROUPS:
        packed_g = _tc_repack(tab2d, cbase, cpg)
        pooled.append(_sc_pooled_embedding_t(packed_g, idx_t, cbase, cpg))
        cbase += cpg
    num = edge_feats[:, _N_CAT:]
    return _tc_dense(pooled, num, W1.T, W2.T)


# speedup vs baseline: 1.0195x; 1.0195x over previous
import functools

import jax
import jax.numpy as jnp
from jax import lax
from jax.experimental import pallas as pl
from jax.experimental.pallas import tpu as pltpu
from jax.experimental.pallas import tpu_sc as plsc

_N_CAT = 26
_N_NUM = 13
_VOCAB = 100001
_EMBED = 32
_HIDDEN = 64
_BATCH = 16384

_GROUPS = (7, 7, 6, 6)

_NLT = 784
_LCH = 12544
_NLCH = 8

_NC = 2
_NS = 16
_NW = _NC * _NS

_IDX_CH = 8192
_NCH = _BATCH // _IDX_CH
_LANES = 16


def _make_repack_body(rpb):
    def _repack_body(in_ref, out_ref):
        for r in range(rpb):
            for k in range(_LCH // 128):
                out_ref[r, k] = in_ref[8 * r:8 * (r + 1), 128 * k:128 * (k + 1)]
    return _repack_body


def _tc_repack(tab2d, cbase, cpg):
    grg = cpg * _EMBED // 8
    rpb = 7 if grg % 8 else 8
    nblk = grg // rpb
    base = cbase * _EMBED // (8 * rpb)
    return pl.pallas_call(
        _make_repack_body(rpb),
        grid=(nblk, _NLCH),
        in_specs=[pl.BlockSpec((8 * rpb, _LCH), lambda i, j: (i + base, j))],
        out_specs=pl.BlockSpec(
            (rpb, _LCH // 128, 8, 128), lambda i, j: (i, j, 0, 0)
        ),
        out_shape=jax.ShapeDtypeStruct((grg, _NLT, 8, 128), jnp.float32),
    )(tab2d)


def _sc_pooled_embedding_t(tab_packed, idx_t, cbase, cpg):
    mesh = plsc.VectorSubcoreMesh(core_axis_name="c", subcore_axis_name="s")

    @functools.partial(
        pl.kernel,
        mesh=mesh,
        compiler_params=pltpu.CompilerParams(
            use_tc_tiling_on_sc=False, needs_layout_passes=False
        ),
        out_type=jax.ShapeDtypeStruct((_EMBED, _BATCH), jnp.float32),
        scratch_types=[
            pltpu.VMEM((_NLT, 1, 128), jnp.float32),
            pltpu.VMEM((_IDX_CH,), jnp.int32),
            pltpu.VMEM((_BATCH,), jnp.float32),
        ],
    )
    def sc_kernel(tab_hbm, idx_hbm, out_hbm, vocab_v, idx_v, acc_v):
        wid = lax.axis_index("s") * _NC + lax.axis_index("c")
        zeros = jnp.zeros((_LANES,), jnp.float32)
        izeros = jnp.zeros((_LANES,), jnp.int32)

        @pl.loop(0, _BATCH // _LANES, unroll=8)
        def _zero(j):
            acc_v[pl.ds(j * _LANES, _LANES)] = zeros

        @pl.loop(0, cpg)
        def _feature(c):
            row = c * _EMBED + wid
            rg = row // 8
            s = row % 8
            pltpu.sync_copy(tab_hbm.at[rg, :, pl.ds(s, 1), :], vocab_v)
            for ch in range(_NCH):
                pltpu.sync_copy(
                    idx_hbm.at[cbase + c, pl.ds(ch * _IDX_CH, _IDX_CH)], idx_v
                )

                @plsc.parallel_loop(0, _IDX_CH // _LANES, unroll=16)
                def _gather(j):
                    ids = idx_v[pl.ds(j * _LANES, _LANES)]
                    lb = lax.shift_right_logical(ids, 7)
                    ln = lax.bitwise_and(ids, 127)
                    vals = plsc.load_gather(vocab_v, [lb, izeros, ln])
                    off = ch * _IDX_CH + j * _LANES
                    plsc.addupdate(acc_v.at[pl.ds(off, _LANES)], vals)

        pltpu.sync_copy(acc_v, out_hbm.at[wid])

    return sc_kernel(tab_packed, idx_t)


def _dense_body(*refs):
    np_ = len(_GROUPS)
    p_refs = refs[:np_]
    num_ref, w1t_ref, w2t_ref, out_ref = refs[np_:]
    w12 = jnp.dot(w1t_ref[...], w2t_ref[...], preferred_element_type=jnp.float32)
    pt = p_refs[0][...]
    for p in p_refs[1:]:
        pt = pt + p[...]
    obj = lax.dot_general(
        pt, w2t_ref[...],
        dimension_numbers=(((0,), (0,)), ((), ())),
        preferred_element_type=jnp.float32,
    )
    out_ref[...] = obj + jnp.dot(num_ref[...], w12, preferred_element_type=jnp.float32)


def _tc_dense(pooled, num, w1t, w2t):
    blk = 2048
    grid = _BATCH // blk
    return pl.pallas_call(
        _dense_body,
        grid=(grid,),
        in_specs=[pl.BlockSpec((_EMBED, blk), lambda i: (0, i))
                  for _ in pooled]
        + [
            pl.BlockSpec((blk, _N_NUM), lambda i: (i, 0)),
            pl.BlockSpec((_N_NUM, _EMBED), lambda i: (0, 0)),
            pl.BlockSpec((_EMBED, _HIDDEN), lambda i: (0, 0)),
        ],
        out_specs=pl.BlockSpec((blk, _HIDDEN), lambda i: (i, 0)),
        out_shape=jax.ShapeDtypeStruct((_BATCH, _HIDDEN), jnp.float32),
    )(*pooled, num, w1t, w2t)


@jax.jit
def kernel(edge_feats, tables, W1, W2):
    tab2d = jnp.transpose(tables, (0, 2, 1)).reshape(_N_CAT * _EMBED, _VOCAB)
    idx_t = jnp.transpose(edge_feats[:, :_N_CAT].astype(jnp.int32), (1, 0))
    pooled = []
    cbase = 0
    for cpg in _GROUPS:
        packed_g = _tc_repack(tab2d, cbase, cpg)
        pooled.append(_sc_pooled_embedding_t(packed_g, idx_t, cbase, cpg))
        cbase += cpg
    num = edge_feats[:, _N_CAT:]
    return _tc_dense(pooled, num, W1.T, W2.T)
